# baseline (speedup 1.0000x reference)
import jax
import jax.numpy as jnp
from jax import lax
from jax.experimental import pallas as pl
from jax.experimental.pallas import tpu as pltpu
from jax.experimental.pallas import tpu_sc as plsc

B = 16384
D = 64
RB = 8
NC = 2
NS = 16
L = 16
NW = NC * NS
BPW = B // NW
NCHUNK = BPW // L


def _mf_body(users_hbm, items_hbm, utab_hbm, itab_hbm, out_hbm,
             uidx, iidx, ubuf, ibuf, outv, sem0, sem1):
    wid = lax.axis_index("s") * NC + lax.axis_index("c")
    base = wid * BPW

    pltpu.sync_copy(users_hbm.at[pl.ds(base, BPW)], uidx)
    pltpu.sync_copy(items_hbm.at[pl.ds(base, BPW)], iidx)

    sems = (sem0, sem1)

    def issue(g, slot, sem):
        ridx_u = uidx[pl.ds(g * L, L)]
        ridx_i = iidx[pl.ds(g * L, L)]
        bu = ridx_u >> 3
        bi = ridx_i >> 3
        for j in range(L):
            pltpu.async_copy(utab_hbm.at[bu[j]], ubuf.at[slot, j], sem)
            pltpu.async_copy(itab_hbm.at[bi[j]], ibuf.at[slot, j], sem)
        return ridx_u & 7, ridx_i & 7

    def finish(g, slot, sem, sub_u, sub_i):
        pltpu.make_async_copy(
            utab_hbm.at[pl.ds(0, L)], ubuf.at[slot], sem).wait()
        pltpu.make_async_copy(
            itab_hbm.at[pl.ds(0, L)], ibuf.at[slot], sem).wait()
        lanes = lax.iota(jnp.int32, L)
        acc = jnp.zeros((L,), jnp.float32)
        for d in range(D):
            col = jnp.full((L,), d, jnp.int32)
            u = plsc.load_gather(ubuf.at[slot], [lanes, sub_u, col])
            v = plsc.load_gather(ibuf.at[slot], [lanes, sub_i, col])
            acc = acc + u * v
        outv[pl.ds(g * L, L)] = acc

    su0, si0 = issue(0, 0, sems[0])

    def step(t, carry):
        su_e, si_e = carry
        g_e = 2 * t
        g_o = 2 * t + 1
        su_o, si_o = issue(g_o, 1, sems[1])
        finish(g_e, 0, sems[0], su_e, si_e)
        su_n, si_n = lax.cond(
            t + 1 < NCHUNK // 2,
            lambda: issue(2 * t + 2, 0, sems[0]),
            lambda: (su_e, si_e),
        )
        finish(g_o, 1, sems[1], su_o, si_o)
        return su_n, si_n

    lax.fori_loop(0, NCHUNK // 2, step, (su0, si0))

    pltpu.sync_copy(outv, out_hbm.at[pl.ds(base, BPW)])


def kernel(users, items, users_table, items_table):
    nu, ni = users_table.shape[0], items_table.shape[0]
    ut3 = users_table.reshape(nu // RB, RB, D)
    it3 = items_table.reshape(ni // RB, RB, D)
    mesh = plsc.VectorSubcoreMesh(core_axis_name="c", subcore_axis_name="s")
    run = pl.kernel(
        _mf_body,
        out_type=jax.ShapeDtypeStruct((B,), jnp.float32),
        mesh=mesh,
        compiler_params=pltpu.CompilerParams(needs_layout_passes=False),
        scratch_types=[
            pltpu.VMEM((BPW,), jnp.int32),
            pltpu.VMEM((BPW,), jnp.int32),
            pltpu.VMEM((2, L, RB, D), jnp.float32),
            pltpu.VMEM((2, L, RB, D), jnp.float32),
            pltpu.VMEM((BPW,), jnp.float32),
            pltpu.SemaphoreType.DMA,
            pltpu.SemaphoreType.DMA,
        ],
    )
    return run(users.astype(jnp.int32), items.astype(jnp.int32), ut3, it3)
